# baseline (speedup 1.0000x reference)
import functools

import jax
import jax.numpy as jnp
from jax import lax
from jax.experimental import pallas as pl
from jax.experimental.pallas import tpu as pltpu
from jax.experimental.pallas import tpu_sc as plsc

HIDDEN = 1024
INTERMEDIATE = 8192
E = 64
VOCAB = 100000
EI = INTERMEDIATE // E
N = 8192

T = 128
NT = N // T
MAX_STEPS = NT + E

NC = 2
NS = 16
NW = NC * NS
L = 16
CHUNK = N // NW
VREGS = CHUNK // L
ROWS = 64
SUB = CHUNK // ROWS

def _sc_mesh():
    return plsc.VectorSubcoreMesh(core_axis_name="c", subcore_axis_name="s",
                                  num_cores=NC, num_subcores=NS)


def _wid():
    return lax.axis_index("s") * NC + lax.axis_index("c")


def _run_ranks(sk, scratch_ref):
    lane = lax.iota(jnp.int32, L)
    scratch_ref[...] = sk
    prev = plsc.load_gather(scratch_ref, [jnp.maximum(lane - 1, 0)])
    nxt = plsc.load_gather(scratch_ref, [jnp.minimum(lane + 1, L - 1)])
    is_start = (lane == 0) | (sk != prev)
    is_last = (lane == L - 1) | (sk != nxt)
    run_start = plsc.cummax(jnp.where(is_start, lane, 0))
    return lane - run_start, is_last


def _route_body(tids_hbm, tte_hbm, e_hbm, lr_hbm, hist_hbm,
                ids_v, tbl_v, e_v, lr_v, hist_v, tmp_v):
    wid = _wid()
    pltpu.sync_copy(tids_hbm.at[pl.ds(wid * CHUNK, CHUNK)], ids_v)
    pltpu.sync_copy(tte_hbm, tbl_v)
    lane = lax.iota(jnp.int32, L)
    for g in range(E // L):
        hist_v[pl.ds(g * L, L)] = jnp.zeros((L,), jnp.int32)
    for v in range(VREGS):
        idv = ids_v[pl.ds(v * L, L)]
        idv = jnp.minimum(jnp.maximum(idv, 0), VOCAB - 1)
        e16 = plsc.load_gather(tbl_v, [idv])
        e_v[pl.ds(v * L, L)] = e16
        sk, sv = plsc.sort_key_val(e16, lane)
        r, is_last = _run_ranks(sk, tmp_v)
        base = plsc.load_gather(hist_v, [sk])
        plsc.store_scatter(tmp_v, [sv], base + r)
        lr_v[pl.ds(v * L, L)] = tmp_v[...]
        plsc.addupdate_scatter(hist_v, [sk], r + 1, mask=is_last)
    pltpu.sync_copy(e_v, e_hbm.at[wid])
    pltpu.sync_copy(lr_v, lr_hbm.at[wid])
    pltpu.sync_copy(hist_v, hist_hbm.at[wid])


def _sc_route(token_ids, token_to_expert):
    f = pl.kernel(
        _route_body,
        out_type=(
            jax.ShapeDtypeStruct((NW, CHUNK), jnp.int32),
            jax.ShapeDtypeStruct((NW, CHUNK), jnp.int32),
            jax.ShapeDtypeStruct((NW, E), jnp.int32),
        ),
        mesh=_sc_mesh(),
        compiler_params=pltpu.CompilerParams(needs_layout_passes=False),
        scratch_types=[
            pltpu.VMEM((CHUNK,), jnp.int32),
            pltpu.VMEM((VOCAB,), jnp.int32),
            pltpu.VMEM((CHUNK,), jnp.int32),
            pltpu.VMEM((CHUNK,), jnp.int32),
            pltpu.VMEM((E,), jnp.int32),
            pltpu.VMEM((L,), jnp.int32),
        ],
    )
    return f(token_ids, token_to_expert)


def _scatter_body(x_hbm, e_hbm, lr_hbm, hist_hbm, sx_hbm, dest_hbm,
                  hv, base_v, e_v, lr_v, dest_v, xbuf, sem):
    wid = _wid()
    pltpu.sync_copy(hist_hbm, hv)
    pltpu.sync_copy(e_hbm.at[wid], e_v)
    pltpu.sync_copy(lr_hbm.at[wid], lr_v)
    carry = jnp.int32(0)
    for g in range(E // L):
        tot = jnp.zeros((L,), jnp.int32)
        mine = jnp.zeros((L,), jnp.int32)
        for t in range(NW):
            h = hv[t, pl.ds(g * L, L)]
            tot = tot + h
            mine = mine + h * (jnp.int32(t) < wid).astype(jnp.int32)
        excl = plsc.cumsum(tot) - tot
        base_v[pl.ds(g * L, L)] = excl + carry + mine
        carry = carry + jnp.sum(tot)
    for v in range(VREGS):
        e16 = e_v[pl.ds(v * L, L)]
        lr16 = lr_v[pl.ds(v * L, L)]
        d16 = plsc.load_gather(base_v, [e16]) + lr16
        dest_v[v // (ROWS // L), pl.ds((v % (ROWS // L)) * L, L)] = d16
    pltpu.sync_copy(dest_v, dest_hbm.at[wid])
    for k in range(SUB):
        pltpu.sync_copy(x_hbm.at[pl.ds(wid * CHUNK + k * ROWS, ROWS)], xbuf)
        pltpu.async_copy(xbuf, sx_hbm.at[dest_v.at[k]], sem).wait()


def _sc_scatter(x, e_chunks, lr, hist):
    f = pl.kernel(
        _scatter_body,
        out_type=(
            jax.ShapeDtypeStruct((N, HIDDEN), jnp.float32),
            jax.ShapeDtypeStruct((NW, SUB, ROWS), jnp.int32),
        ),
        mesh=_sc_mesh(),
        compiler_params=pltpu.CompilerParams(needs_layout_passes=False),
        scratch_types=[
            pltpu.VMEM((NW, E), jnp.int32),
            pltpu.VMEM((E,), jnp.int32),
            pltpu.VMEM((CHUNK,), jnp.int32),
            pltpu.VMEM((CHUNK,), jnp.int32),
            pltpu.VMEM((SUB, ROWS), jnp.int32),
            pltpu.VMEM((ROWS, HIDDEN), jnp.float32),
            pltpu.SemaphoreType.DMA,
        ],
    )
    return f(x, e_chunks, lr, hist)


def _unsort_body(os_hbm, dest_hbm, fin_hbm, dest_v, buf, sem):
    wid = _wid()
    pltpu.sync_copy(dest_hbm.at[wid], dest_v)
    for k in range(SUB):
        pltpu.async_copy(os_hbm.at[dest_v.at[k]], buf, sem).wait()
        pltpu.sync_copy(buf, fin_hbm.at[pl.ds(wid * CHUNK + k * ROWS, ROWS)])


def _sc_unsort(out_sorted, dest):
    f = pl.kernel(
        _unsort_body,
        out_type=jax.ShapeDtypeStruct((N, HIDDEN), jnp.float32),
        mesh=_sc_mesh(),
        compiler_params=pltpu.CompilerParams(needs_layout_passes=False),
        scratch_types=[
            pltpu.VMEM((SUB, ROWS), jnp.int32),
            pltpu.VMEM((ROWS, HIDDEN), jnp.float32),
            pltpu.SemaphoreType.DMA,
        ],
    )
    return f(out_sorted, dest)


def _gmm_body(meta_ref, x_ref, gu_ref, dn_ref, o_ref):
    w = pl.program_id(0)
    rs = meta_ref[2, w]
    re_ = meta_ref[3, w]
    first = meta_ref[4, w]
    x = x_ref[...]
    gu = jnp.dot(x, gu_ref[0], preferred_element_type=jnp.float32)
    gate = gu[:, :EI]
    up = gu[:, EI:]
    inter = gate * jax.nn.sigmoid(gate) * up
    part = jnp.dot(inter, dn_ref[0], preferred_element_type=jnp.float32)
    rows = lax.broadcasted_iota(jnp.int32, (T, 1), 0)
    mask = (rows >= rs) & (rows < re_)

    @pl.when(first == 1)
    def _():
        o_ref[...] = jnp.where(mask, part, 0.0)

    @pl.when(first == 0)
    def _():
        o_ref[...] = jnp.where(mask, part, o_ref[...])


def _grouped_mlp(sorted_x, gate_up_proj, down_proj, meta):
    grid_spec = pltpu.PrefetchScalarGridSpec(
        num_scalar_prefetch=1,
        grid=(MAX_STEPS,),
        in_specs=[
            pl.BlockSpec((T, HIDDEN), lambda w, m: (m[0, w], 0)),
            pl.BlockSpec((1, HIDDEN, 2 * EI), lambda w, m: (m[1, w], 0, 0)),
            pl.BlockSpec((1, EI, HIDDEN), lambda w, m: (m[1, w], 0, 0)),
        ],
        out_specs=pl.BlockSpec((T, HIDDEN), lambda w, m: (m[0, w], 0)),
    )
    return pl.pallas_call(
        _gmm_body,
        grid_spec=grid_spec,
        out_shape=jax.ShapeDtypeStruct((N, HIDDEN), jnp.float32),
    )(meta, sorted_x, gate_up_proj, down_proj)


def _step_metadata(counts):
    ends = jnp.cumsum(counts)
    starts = ends - counts
    first_tile = starts // T
    last_tile = jnp.maximum(ends - 1, 0) // T
    nsteps = jnp.where(counts > 0, last_tile - first_tile + 1, 0)
    inc = jnp.cumsum(nsteps)
    step_off = inc - nsteps
    total = inc[-1]
    w = jnp.arange(MAX_STEPS, dtype=jnp.int32)
    e_w = jnp.searchsorted(inc, w, side="right").astype(jnp.int32)
    e_w = jnp.minimum(e_w, E - 1)
    j = w - step_off[e_w]
    tile_w = first_tile[e_w] + j
    rs = jnp.maximum(starts[e_w] - tile_w * T, 0)
    re_ = jnp.minimum(ends[e_w] - tile_w * T, T)
    valid = w < total
    tile_w = jnp.where(valid, tile_w, NT - 1)
    rs = jnp.where(valid, rs, 0)
    re_ = jnp.where(valid, re_, 0)
    prev_tile = jnp.concatenate([jnp.full((1,), -1, jnp.int32), tile_w[:-1]])
    first = (tile_w != prev_tile).astype(jnp.int32)
    return jnp.stack([tile_w.astype(jnp.int32), e_w, rs.astype(jnp.int32),
                      re_.astype(jnp.int32), first])


def kernel(x, token_ids, gate_up_proj, down_proj, token_to_expert):
    e_chunks, lr, hist = _sc_route(token_ids.astype(jnp.int32),
                                   token_to_expert.astype(jnp.int32))
    sorted_x, dest = _sc_scatter(x, e_chunks, lr, hist)
    counts = hist.sum(axis=0).astype(jnp.int32)
    meta = _step_metadata(counts)
    out_sorted = _grouped_mlp(sorted_x, gate_up_proj, down_proj, meta)
    return _sc_unsort(out_sorted, dest)
